# baseline (speedup 1.0000x reference)
import functools

import jax
import jax.numpy as jnp
from jax import lax
from jax.experimental import pallas as pl
from jax.experimental.pallas import tpu as pltpu
from jax.experimental.pallas import tpu_sc as plsc

INPUT_DIM = 768
HIDDEN = INPUT_DIM // 2
NUM_EXPERTS = 64
N_TOKENS = 32768
BT = 4096

NC = 2
NS = 16
NW = NC * NS
TPW = N_TOKENS // NW
L = 16


def _logits_kernel(x_ref, w1_ref, b1_ref, w2_ref, b2_ref, lt_ref):
    h = jnp.dot(x_ref[:], w1_ref[:], preferred_element_type=jnp.float32)
    h = jnp.maximum(h + b1_ref[:], 0.0)
    logits = jnp.dot(h, w2_ref[:], preferred_element_type=jnp.float32)
    logits = logits + b2_ref[:]
    lt_ref[:] = logits.T


def _tc_logits(x, W1, b1, W2, b2):
    n = x.shape[0]
    return pl.pallas_call(
        _logits_kernel,
        grid=(n // BT,),
        in_specs=[
            pl.BlockSpec((BT, INPUT_DIM), lambda i: (i, 0)),
            pl.BlockSpec((INPUT_DIM, HIDDEN), lambda i: (0, 0)),
            pl.BlockSpec((1, HIDDEN), lambda i: (0, 0)),
            pl.BlockSpec((HIDDEN, NUM_EXPERTS), lambda i: (0, 0)),
            pl.BlockSpec((1, NUM_EXPERTS), lambda i: (0, 0)),
        ],
        out_specs=pl.BlockSpec((NUM_EXPERTS, BT), lambda i: (0, i)),
        out_shape=jax.ShapeDtypeStruct((NUM_EXPERTS, n), jnp.float32),
        compiler_params=pltpu.CompilerParams(
            dimension_semantics=("parallel",),
        ),
    )(x, W1, b1.reshape(1, HIDDEN), W2, b2.reshape(1, NUM_EXPERTS))


def _sc_body(lt_hbm, ow_hbm, oi_hbm, vm, ow, oi):
    wid = lax.axis_index("s") * NC + lax.axis_index("c")
    base = wid * TPW
    pltpu.sync_copy(lt_hbm.at[:, pl.ds(base, TPW)], vm)

    def outer(t, _):
        tt = t * L

        def inner(e, carry):
            m1, i1, m2, i2 = carry
            v = vm[e, pl.ds(tt, L)]
            ev = jnp.full((L,), e, jnp.int32)
            gt1 = v > m1
            gt2 = v > m2
            m2n = jnp.where(gt1, m1, jnp.where(gt2, v, m2))
            i2n = jnp.where(gt1, i1, jnp.where(gt2, ev, i2))
            m1n = jnp.where(gt1, v, m1)
            i1n = jnp.where(gt1, ev, i1)
            return m1n, i1n, m2n, i2n

        neg = jnp.full((L,), -jnp.inf, jnp.float32)
        zero = jnp.zeros((L,), jnp.int32)
        m1, i1, m2, i2 = lax.fori_loop(0, NUM_EXPERTS, inner,
                                       (neg, zero, neg, zero))
        e2 = jnp.exp(m2 - m1)
        inv = 1.0 / (1.0 + e2)
        ow[0, pl.ds(tt, L)] = inv
        ow[1, pl.ds(tt, L)] = e2 * inv
        oi[0, pl.ds(tt, L)] = i1
        oi[1, pl.ds(tt, L)] = i2
        return 0

    lax.fori_loop(0, TPW // L, outer, 0)
    pltpu.sync_copy(ow, ow_hbm.at[:, pl.ds(base, TPW)])
    pltpu.sync_copy(oi, oi_hbm.at[:, pl.ds(base, TPW)])


_sc_topk = functools.partial(
    pl.kernel,
    out_type=[
        jax.ShapeDtypeStruct((2, N_TOKENS), jnp.float32),
        jax.ShapeDtypeStruct((2, N_TOKENS), jnp.int32),
    ],
    mesh=plsc.VectorSubcoreMesh(
        core_axis_name="c", subcore_axis_name="s",
        num_cores=NC, num_subcores=NS),
    scratch_types=[
        pltpu.VMEM((NUM_EXPERTS, TPW), jnp.float32),
        pltpu.VMEM((2, TPW), jnp.float32),
        pltpu.VMEM((2, TPW), jnp.int32),
    ],
)(_sc_body)


@jax.jit
def kernel(x, W1, b1, W2, b2):
    lt = _tc_logits(x, W1, b1, W2, b2)
    ow, oi = _sc_topk(lt)
    return (ow.T, oi.T)
